# baseline (speedup 1.0000x reference)
import jax
import jax.numpy as jnp
from jax.experimental import pallas as pl
from jax.experimental.pallas import tpu as pltpu

_B = 4
_S = 8192
_D = 1024
_SBLK = 512
_NWB = _S // _SBLK


def _body(x_ref, w_ref, o_ref):
    o_ref[...] = x_ref[...] + w_ref[...]


@jax.jit
def _pos_add(x, w):
    x2 = x.reshape(_B * _S, _D)
    out = pl.pallas_call(
        _body,
        grid=(_NWB, _B),
        in_specs=[
            pl.BlockSpec((_SBLK, _D), lambda i, j: (j * _NWB + i, 0)),
            pl.BlockSpec((_SBLK, _D), lambda i, j: (i, 0)),
        ],
        out_specs=pl.BlockSpec((_SBLK, _D), lambda i, j: (j * _NWB + i, 0)),
        out_shape=jax.ShapeDtypeStruct((_B * _S, _D), jnp.float32),
        compiler_params=pltpu.CompilerParams(
            dimension_semantics=("arbitrary", "arbitrary"),
        ),
    )(x2, w)
    return out.reshape(_B, _S, _D)


def kernel(x, weight):
    return _pos_add(x, weight)


# speedup vs baseline: 1.1499x; 1.1499x over previous
import jax
import jax.numpy as jnp
from jax.experimental import pallas as pl
from jax.experimental.pallas import tpu as pltpu

_B = 4
_S = 8192
_D = 1024
_SBLK = 512


def _body(x_ref, w_ref, o_ref):
    o_ref[...] = x_ref[...] + w_ref[...][None, :, :]


@jax.jit
def _pos_add(x, w):
    return pl.pallas_call(
        _body,
        grid=(_S // _SBLK,),
        in_specs=[
            pl.BlockSpec((_B, _SBLK, _D), lambda i: (0, i, 0)),
            pl.BlockSpec((_SBLK, _D), lambda i: (i, 0)),
        ],
        out_specs=pl.BlockSpec((_B, _SBLK, _D), lambda i: (0, i, 0)),
        out_shape=jax.ShapeDtypeStruct((_B, _S, _D), jnp.float32),
        compiler_params=pltpu.CompilerParams(
            dimension_semantics=("arbitrary",),
        ),
    )(x, w)


def kernel(x, weight):
    return _pos_add(x, weight)
